# baseline (speedup 1.0000x reference)
import functools

import jax
import jax.numpy as jnp
from jax import lax
from jax.experimental import pallas as pl
from jax.experimental.pallas import tpu as pltpu
from jax.experimental.pallas import tpu_sc as plsc

_N = 10000
_E = 320000
_D = 128
_H = 128

_NC = 2
_NS = 16
_NW = _NC * _NS
_CH = 48
_NCHUNK = 216
_GRP = 24
_NGRP = _NCHUNK // _GRP
_EPW = _CH * _NCHUNK
_EPAD = _NW * _EPW
_RPT = _N // _NS

_MB = 1000


def _mm0_body(x_ref, w_ref, o_ref):
    o_ref[...] = jnp.dot(x_ref[...], w_ref[...],
                         preferred_element_type=jnp.float32)


def _tc_linear0(x, w):
    return pl.pallas_call(
        _mm0_body,
        grid=(_N // _MB,),
        in_specs=[
            pl.BlockSpec((_MB, _D), lambda i: (i, 0)),
            pl.BlockSpec((_D, _H), lambda i: (0, 0)),
        ],
        out_specs=pl.BlockSpec((_MB, _H), lambda i: (i, 0)),
        out_shape=jax.ShapeDtypeStruct((_N, _H), jnp.float32),
    )(x, w)


def _fuse_body(p_ref, b_ref, a_ref, w_ref, o_ref):
    s = p_ref[0] + p_ref[1] + b_ref[...]
    a = a_ref[0, 0]
    x = jnp.where(s >= 0, s, a * s)
    o_ref[...] = jnp.dot(x, w_ref[...], preferred_element_type=jnp.float32)


def _tc_fuse(p, b, a, w):
    return pl.pallas_call(
        _fuse_body,
        grid=(_N // _MB,),
        in_specs=[
            pl.BlockSpec((2, _MB, _H), lambda i: (0, i, 0)),
            pl.BlockSpec((1, _H), lambda i: (0, 0)),
            pl.BlockSpec((1, 1), lambda i: (0, 0)),
            pl.BlockSpec((_H, _H), lambda i: (0, 0)),
        ],
        out_specs=pl.BlockSpec((_MB, _H), lambda i: (i, 0)),
        out_shape=jax.ShapeDtypeStruct((_N, _H), jnp.float32),
    )(p, b.reshape(1, _H), a.reshape(1, 1), w)


def _final_body(p_ref, b_ref, a_ref, o_ref):
    s = p_ref[0] + p_ref[1] + b_ref[...]
    a = a_ref[0, 0]
    o_ref[...] = jnp.where(s >= 0, s, a * s)


def _tc_final(p, b, a):
    return pl.pallas_call(
        _final_body,
        grid=(_N // _MB,),
        in_specs=[
            pl.BlockSpec((2, _MB, _H), lambda i: (0, i, 0)),
            pl.BlockSpec((1, _H), lambda i: (0, 0)),
            pl.BlockSpec((1, 1), lambda i: (0, 0)),
        ],
        out_specs=pl.BlockSpec((_MB, _H), lambda i: (i, 0)),
        out_shape=jax.ShapeDtypeStruct((_N, _H), jnp.float32),
    )(p, b.reshape(1, _H), a.reshape(1, 1))


def _scale_chunk(g_ref, s_ref, w_ref, c):
    iot = lax.broadcasted_iota(jnp.int32, (16,), 0)
    groups = []
    for g in range(_CH // 16):
        groups.append((iot + g * 16, w_ref[c, pl.ds(g * 16, 16)]))

    for col in range(_H):
        cidx = jnp.full((16,), col, jnp.int32)
        for ridx, wv in groups:
            vals = plsc.load_gather(g_ref, [ridx, cidx])
            plsc.store_scatter(s_ref, [ridx, cidx], vals * wv)


def _sc_spmm_body(fts_hbm, src_hbm, dst_hbm, w_hbm, out_hbm,
                  accum, src_g, dst_g, w_g,
                  gbuf0, gbuf1, sbuf0, sbuf1,
                  gsem0, gsem1, ssem0, ssem1):
    cid = lax.axis_index("c")
    sid = lax.axis_index("s")
    wid = sid * _NC + cid

    def stage_group(gi):
        gsl = pl.ds(gi * _GRP, _GRP)
        pltpu.sync_copy(src_hbm.at[wid, gsl], src_g)
        pltpu.sync_copy(dst_hbm.at[wid, gsl], dst_g)
        pltpu.sync_copy(w_hbm.at[wid, gsl], w_g)

    def wait_gather(buf, sem):
        pltpu.make_async_copy(fts_hbm.at[src_g.at[0]], buf, sem).wait()

    def wait_scatter(buf, sem):
        pltpu.make_async_copy(buf, accum.at[dst_g.at[0]], sem).wait()

    def zbody(r, carry):
        for j in range(_H // 16):
            sbuf0[r, pl.ds(j * 16, 16)] = jnp.zeros((16,), jnp.float32)
        return carry

    lax.fori_loop(0, _CH, zbody, 0)
    nfull = _RPT // _CH
    for i in range(nfull):
        pltpu.sync_copy(sbuf0,
                        accum.at[pl.ds(sid * _RPT + i * _CH, _CH)])
    rem = _RPT - nfull * _CH
    if rem:
        pltpu.sync_copy(sbuf0.at[pl.ds(0, rem)],
                        accum.at[pl.ds(sid * _RPT + nfull * _CH, rem)])

    plsc.subcore_barrier()

    ppg = _GRP // 2
    npairs = _NCHUNK // 2

    def step(t, carry):
        gi = t // ppg
        tl = t - gi * ppg
        la = 2 * tl
        lb = la + 1

        @pl.when(tl == 0)
        def _():
            @pl.when(t > 0)
            def _():
                wait_scatter(sbuf0, ssem0)
                wait_scatter(sbuf1, ssem1)

            stage_group(gi)
            pltpu.async_copy(fts_hbm.at[src_g.at[0]], gbuf0, gsem0)
            pltpu.async_copy(fts_hbm.at[src_g.at[1]], gbuf1, gsem1)

        wait_gather(gbuf0, gsem0)

        @pl.when(tl > 0)
        def _():
            wait_scatter(sbuf0, ssem0)

        _scale_chunk(gbuf0, sbuf0, w_g, la)

        @pl.when(tl < ppg - 1)
        def _():
            pltpu.async_copy(fts_hbm.at[src_g.at[la + 2]], gbuf0, gsem0)

        pltpu.async_copy(sbuf0, accum.at[dst_g.at[la]], ssem0, add=True)

        wait_gather(gbuf1, gsem1)

        @pl.when(tl > 0)
        def _():
            wait_scatter(sbuf1, ssem1)

        _scale_chunk(gbuf1, sbuf1, w_g, lb)

        @pl.when(tl < ppg - 1)
        def _():
            pltpu.async_copy(fts_hbm.at[src_g.at[lb + 2]], gbuf1, gsem1)

        pltpu.async_copy(sbuf1, accum.at[dst_g.at[lb]], ssem1, add=True)
        return carry

    lax.fori_loop(0, npairs, step, 0)

    wait_scatter(sbuf0, ssem0)
    wait_scatter(sbuf1, ssem1)

    plsc.subcore_barrier()

    nchunks = _N // 200
    for c in range((nchunks + _NS - 1) // _NS):
        chunk = sid + c * _NS

        @pl.when(chunk < nchunks)
        def _():
            r0 = chunk * 200
            pltpu.sync_copy(accum.at[pl.ds(r0, 200)],
                            out_hbm.at[cid, pl.ds(r0, 200)])


@functools.partial(jax.jit, static_argnames=())
def _sc_spmm(fts, srcb, dstb, wb):
    mesh = plsc.VectorSubcoreMesh(core_axis_name="c", subcore_axis_name="s",
                                  num_cores=_NC, num_subcores=_NS)
    f = pl.kernel(
        _sc_spmm_body,
        out_type=jax.ShapeDtypeStruct((_NC, _N, _H), jnp.float32),
        mesh=mesh,
        compiler_params=pltpu.CompilerParams(needs_layout_passes=False),
        scratch_types=[
            pltpu.VMEM_SHARED((_N, _H), jnp.float32),
            pltpu.VMEM((_GRP, _CH), jnp.int32),
            pltpu.VMEM((_GRP, _CH), jnp.int32),
            pltpu.VMEM((_GRP, _CH), jnp.float32),
            pltpu.VMEM((_CH, _H), jnp.float32),
            pltpu.VMEM((_CH, _H), jnp.float32),
            pltpu.VMEM((_CH, _H), jnp.float32),
            pltpu.VMEM((_CH, _H), jnp.float32),
            pltpu.SemaphoreType.DMA,
            pltpu.SemaphoreType.DMA,
            pltpu.SemaphoreType.DMA,
            pltpu.SemaphoreType.DMA,
        ],
    )
    return f(fts, srcb, dstb, wb)


def kernel(seq, edge_index, edge_weight, W0, b0, a0, W1, b1, a1, W2, b2, a2,
           sparse=1):
    x = seq[0]
    src = edge_index[0]
    dst = edge_index[1]

    pad = _EPAD - _E
    zi = jnp.zeros((pad,), jnp.int32)
    srcb = jnp.concatenate([src, zi]).reshape(_NW, _NCHUNK, _CH)
    dstb = jnp.concatenate([dst, zi]).reshape(_NW, _NCHUNK, _CH)
    wb = jnp.concatenate([edge_weight, jnp.zeros((pad,), jnp.float32)]
                         ).reshape(_NW, _NCHUNK, _CH)

    a0 = jnp.asarray(a0, jnp.float32)
    a1 = jnp.asarray(a1, jnp.float32)
    a2 = jnp.asarray(a2, jnp.float32)

    y = _tc_linear0(x, W0)
    p = _sc_spmm(y, srcb, dstb, wb)
    y = _tc_fuse(p, b0, a0, W1)
    p = _sc_spmm(y, srcb, dstb, wb)
    y = _tc_fuse(p, b1, a1, W2)
    p = _sc_spmm(y, srcb, dstb, wb)
    out = _tc_final(p, b2, a2)
    return out[None]


# speedup vs baseline: 2.6324x; 2.6324x over previous
import functools

import jax
import jax.numpy as jnp
from jax import lax
from jax.experimental import pallas as pl
from jax.experimental.pallas import tpu as pltpu
from jax.experimental.pallas import tpu_sc as plsc

_N = 10000
_E = 320000
_D = 128
_H = 128

_NC = 2
_NS = 16
_NW = _NC * _NS
_CH = 48
_NCHUNK = 216
_GRP = 24
_NGRP = _NCHUNK // _GRP
_EPW = _CH * _NCHUNK
_EPAD = _NW * _EPW
_RPT = _N // _NS

_MB = 1000


def _mm0_body(x_ref, w_ref, o_ref):
    o_ref[...] = jnp.dot(x_ref[...], w_ref[...],
                         preferred_element_type=jnp.float32)


def _tc_linear0(x, w):
    return pl.pallas_call(
        _mm0_body,
        grid=(_N // _MB,),
        in_specs=[
            pl.BlockSpec((_MB, _D), lambda i: (i, 0)),
            pl.BlockSpec((_D, _H), lambda i: (0, 0)),
        ],
        out_specs=pl.BlockSpec((_MB, _H), lambda i: (i, 0)),
        out_shape=jax.ShapeDtypeStruct((_N, _H), jnp.float32),
    )(x, w)


def _fuse_body(p_ref, b_ref, a_ref, w_ref, o_ref):
    s = p_ref[0] + p_ref[1] + b_ref[...]
    a = a_ref[0, 0]
    x = jnp.where(s >= 0, s, a * s)
    o_ref[...] = jnp.dot(x, w_ref[...], preferred_element_type=jnp.float32)


def _tc_fuse(p, b, a, w):
    return pl.pallas_call(
        _fuse_body,
        grid=(_N // _MB,),
        in_specs=[
            pl.BlockSpec((2, _MB, _H), lambda i: (0, i, 0)),
            pl.BlockSpec((1, _H), lambda i: (0, 0)),
            pl.BlockSpec((1, 1), lambda i: (0, 0)),
            pl.BlockSpec((_H, _H), lambda i: (0, 0)),
        ],
        out_specs=pl.BlockSpec((_MB, _H), lambda i: (i, 0)),
        out_shape=jax.ShapeDtypeStruct((_N, _H), jnp.float32),
    )(p, b.reshape(1, _H), a.reshape(1, 1), w)


def _final_body(p_ref, b_ref, a_ref, o_ref):
    s = p_ref[0] + p_ref[1] + b_ref[...]
    a = a_ref[0, 0]
    o_ref[...] = jnp.where(s >= 0, s, a * s)


def _tc_final(p, b, a):
    return pl.pallas_call(
        _final_body,
        grid=(_N // _MB,),
        in_specs=[
            pl.BlockSpec((2, _MB, _H), lambda i: (0, i, 0)),
            pl.BlockSpec((1, _H), lambda i: (0, 0)),
            pl.BlockSpec((1, 1), lambda i: (0, 0)),
        ],
        out_specs=pl.BlockSpec((_MB, _H), lambda i: (i, 0)),
        out_shape=jax.ShapeDtypeStruct((_N, _H), jnp.float32),
    )(p, b.reshape(1, _H), a.reshape(1, 1))


def _scale_chunk(g_ref, s_ref, w_ref, c):
    for g in range(_CH // 16):
        wv = w_ref[c, pl.ds(g * 16, 16)]
        for i in range(16):
            e = g * 16 + i
            w = wv[i]
            for j in range(_H // 16):
                sl = pl.ds(j * 16, 16)
                s_ref[e, sl] = g_ref[e, sl] * w


def _sc_spmm_body(fts_hbm, src_hbm, dst_hbm, w_hbm, out_hbm,
                  accum, src_g, dst_g, w_g,
                  gbuf0, gbuf1, sbuf0, sbuf1,
                  gsem0, gsem1, ssem0, ssem1):
    cid = lax.axis_index("c")
    sid = lax.axis_index("s")
    wid = sid * _NC + cid

    def stage_group(gi):
        gsl = pl.ds(gi * _GRP, _GRP)
        pltpu.sync_copy(src_hbm.at[wid, gsl], src_g)
        pltpu.sync_copy(dst_hbm.at[wid, gsl], dst_g)
        pltpu.sync_copy(w_hbm.at[wid, gsl], w_g)

    def wait_gather(buf, sem):
        pltpu.make_async_copy(fts_hbm.at[src_g.at[0]], buf, sem).wait()

    def wait_scatter(buf, sem):
        pltpu.make_async_copy(buf, accum.at[dst_g.at[0]], sem).wait()

    def zbody(r, carry):
        for j in range(_H // 16):
            sbuf0[r, pl.ds(j * 16, 16)] = jnp.zeros((16,), jnp.float32)
        return carry

    lax.fori_loop(0, _CH, zbody, 0)
    nfull = _RPT // _CH
    for i in range(nfull):
        pltpu.sync_copy(sbuf0,
                        accum.at[pl.ds(sid * _RPT + i * _CH, _CH)])
    rem = _RPT - nfull * _CH
    if rem:
        pltpu.sync_copy(sbuf0.at[pl.ds(0, rem)],
                        accum.at[pl.ds(sid * _RPT + nfull * _CH, rem)])

    plsc.subcore_barrier()

    ppg = _GRP // 2
    npairs = _NCHUNK // 2

    def step(t, carry):
        gi = t // ppg
        tl = t - gi * ppg
        la = 2 * tl
        lb = la + 1

        @pl.when(tl == 0)
        def _():
            @pl.when(t > 0)
            def _():
                wait_scatter(sbuf0, ssem0)
                wait_scatter(sbuf1, ssem1)

            stage_group(gi)
            pltpu.async_copy(fts_hbm.at[src_g.at[0]], gbuf0, gsem0)
            pltpu.async_copy(fts_hbm.at[src_g.at[1]], gbuf1, gsem1)

        wait_gather(gbuf0, gsem0)

        @pl.when(tl > 0)
        def _():
            wait_scatter(sbuf0, ssem0)

        _scale_chunk(gbuf0, sbuf0, w_g, la)

        @pl.when(tl < ppg - 1)
        def _():
            pltpu.async_copy(fts_hbm.at[src_g.at[la + 2]], gbuf0, gsem0)

        pltpu.async_copy(sbuf0, accum.at[dst_g.at[la]], ssem0, add=True)

        wait_gather(gbuf1, gsem1)

        @pl.when(tl > 0)
        def _():
            wait_scatter(sbuf1, ssem1)

        _scale_chunk(gbuf1, sbuf1, w_g, lb)

        @pl.when(tl < ppg - 1)
        def _():
            pltpu.async_copy(fts_hbm.at[src_g.at[lb + 2]], gbuf1, gsem1)

        pltpu.async_copy(sbuf1, accum.at[dst_g.at[lb]], ssem1, add=True)
        return carry

    lax.fori_loop(0, npairs, step, 0)

    wait_scatter(sbuf0, ssem0)
    wait_scatter(sbuf1, ssem1)

    plsc.subcore_barrier()

    nchunks = _N // 200
    for c in range((nchunks + _NS - 1) // _NS):
        chunk = sid + c * _NS

        @pl.when(chunk < nchunks)
        def _():
            r0 = chunk * 200
            pltpu.sync_copy(accum.at[pl.ds(r0, 200)],
                            out_hbm.at[cid, pl.ds(r0, 200)])


@functools.partial(jax.jit, static_argnames=())
def _sc_spmm(fts, srcb, dstb, wb):
    mesh = plsc.VectorSubcoreMesh(core_axis_name="c", subcore_axis_name="s",
                                  num_cores=_NC, num_subcores=_NS)
    f = pl.kernel(
        _sc_spmm_body,
        out_type=jax.ShapeDtypeStruct((_NC, _N, _H), jnp.float32),
        mesh=mesh,
        compiler_params=pltpu.CompilerParams(needs_layout_passes=False),
        scratch_types=[
            pltpu.VMEM_SHARED((_N, _H), jnp.float32),
            pltpu.VMEM((_GRP, _CH), jnp.int32),
            pltpu.VMEM((_GRP, _CH), jnp.int32),
            pltpu.VMEM((_GRP, _CH), jnp.float32),
            pltpu.VMEM((_CH, _H), jnp.float32),
            pltpu.VMEM((_CH, _H), jnp.float32),
            pltpu.VMEM((_CH, _H), jnp.float32),
            pltpu.VMEM((_CH, _H), jnp.float32),
            pltpu.SemaphoreType.DMA,
            pltpu.SemaphoreType.DMA,
            pltpu.SemaphoreType.DMA,
            pltpu.SemaphoreType.DMA,
        ],
    )
    return f(fts, srcb, dstb, wb)


def kernel(seq, edge_index, edge_weight, W0, b0, a0, W1, b1, a1, W2, b2, a2,
           sparse=1):
    x = seq[0]
    src = edge_index[0]
    dst = edge_index[1]

    pad = _EPAD - _E
    zi = jnp.zeros((pad,), jnp.int32)
    srcb = jnp.concatenate([src, zi]).reshape(_NW, _NCHUNK, _CH)
    dstb = jnp.concatenate([dst, zi]).reshape(_NW, _NCHUNK, _CH)
    wb = jnp.concatenate([edge_weight, jnp.zeros((pad,), jnp.float32)]
                         ).reshape(_NW, _NCHUNK, _CH)

    a0 = jnp.asarray(a0, jnp.float32)
    a1 = jnp.asarray(a1, jnp.float32)
    a2 = jnp.asarray(a2, jnp.float32)

    y = _tc_linear0(x, W0)
    p = _sc_spmm(y, srcb, dstb, wb)
    y = _tc_fuse(p, b0, a0, W1)
    p = _sc_spmm(y, srcb, dstb, wb)
    y = _tc_fuse(p, b1, a1, W2)
    p = _sc_spmm(y, srcb, dstb, wb)
    out = _tc_final(p, b2, a2)
    return out[None]


# speedup vs baseline: 3.5041x; 1.3312x over previous
import functools

import jax
import jax.numpy as jnp
from jax import lax
from jax.experimental import pallas as pl
from jax.experimental.pallas import tpu as pltpu
from jax.experimental.pallas import tpu_sc as plsc

_N = 10000
_E = 320000
_D = 128
_H = 128

_NC = 2
_NS = 16
_NW = _NC * _NS
_CH = 32
_NCHUNK = 320
_GRP = 32
_NGRP = _NCHUNK // _GRP
_EPW = _CH * _NCHUNK
_EPAD = _NW * _EPW
_RPT = _N // _NS

_MB = 1000


def _mm0_body(x_ref, w_ref, o_ref):
    o_ref[...] = jnp.dot(x_ref[...], w_ref[...],
                         preferred_element_type=jnp.float32)


def _tc_linear0(x, w):
    return pl.pallas_call(
        _mm0_body,
        grid=(_N // _MB,),
        in_specs=[
            pl.BlockSpec((_MB, _D), lambda i: (i, 0)),
            pl.BlockSpec((_D, _H), lambda i: (0, 0)),
        ],
        out_specs=pl.BlockSpec((_MB, _H), lambda i: (i, 0)),
        out_shape=jax.ShapeDtypeStruct((_N, _H), jnp.float32),
    )(x, w)


def _fuse_body(p_ref, b_ref, a_ref, w_ref, o_ref):
    s = p_ref[0] + p_ref[1] + b_ref[...]
    a = a_ref[0, 0]
    x = jnp.where(s >= 0, s, a * s)
    o_ref[...] = jnp.dot(x, w_ref[...], preferred_element_type=jnp.float32)


def _tc_fuse(p, b, a, w):
    return pl.pallas_call(
        _fuse_body,
        grid=(_N // _MB,),
        in_specs=[
            pl.BlockSpec((2, _MB, _H), lambda i: (0, i, 0)),
            pl.BlockSpec((1, _H), lambda i: (0, 0)),
            pl.BlockSpec((1, 1), lambda i: (0, 0)),
            pl.BlockSpec((_H, _H), lambda i: (0, 0)),
        ],
        out_specs=pl.BlockSpec((_MB, _H), lambda i: (i, 0)),
        out_shape=jax.ShapeDtypeStruct((_N, _H), jnp.float32),
    )(p, b.reshape(1, _H), a.reshape(1, 1), w)


def _final_body(p_ref, b_ref, a_ref, o_ref):
    s = p_ref[0] + p_ref[1] + b_ref[...]
    a = a_ref[0, 0]
    o_ref[...] = jnp.where(s >= 0, s, a * s)


def _tc_final(p, b, a):
    return pl.pallas_call(
        _final_body,
        grid=(_N // _MB,),
        in_specs=[
            pl.BlockSpec((2, _MB, _H), lambda i: (0, i, 0)),
            pl.BlockSpec((1, _H), lambda i: (0, 0)),
            pl.BlockSpec((1, 1), lambda i: (0, 0)),
        ],
        out_specs=pl.BlockSpec((_MB, _H), lambda i: (i, 0)),
        out_shape=jax.ShapeDtypeStruct((_N, _H), jnp.float32),
    )(p, b.reshape(1, _H), a.reshape(1, 1))


def _scale_chunk(g_ref, s_ref, w_ref, c):
    for g in range(_CH // 16):
        wv = w_ref[c, pl.ds(g * 16, 16)]
        for i in range(16):
            e = g * 16 + i
            w = wv[i]
            for j in range(_H // 16):
                sl = pl.ds(j * 16, 16)
                s_ref[e, sl] = g_ref[e, sl] * w


def _sc_spmm_body(fts_hbm, src_hbm, dst_hbm, w_hbm, out_hbm,
                  accum, src_g, dst_g, w_g,
                  gb0, gb1, gb2, gb3, sb0, sb1,
                  gm0, gm1, gm2, gm3, sm0, sm1):
    cid = lax.axis_index("c")
    sid = lax.axis_index("s")
    wid = sid * _NC + cid

    gbufs = (gb0, gb1, gb2, gb3)
    gsems = (gm0, gm1, gm2, gm3)
    sbufs = (sb0, sb1)
    ssems = (sm0, sm1)

    def stage_group(gi):
        gsl = pl.ds(gi * _GRP, _GRP)
        pltpu.sync_copy(src_hbm.at[wid, gsl], src_g)
        pltpu.sync_copy(dst_hbm.at[wid, gsl], dst_g)
        pltpu.sync_copy(w_hbm.at[wid, gsl], w_g)

    def wait_gather(j):
        pltpu.make_async_copy(fts_hbm.at[src_g.at[0]], gbufs[j],
                              gsems[j]).wait()

    def wait_scatter(j):
        pltpu.make_async_copy(sbufs[j], accum.at[dst_g.at[0]],
                              ssems[j]).wait()

    def zbody(r, carry):
        for j in range(_H // 16):
            sb0[r, pl.ds(j * 16, 16)] = jnp.zeros((16,), jnp.float32)
        return carry

    lax.fori_loop(0, _CH, zbody, 0)
    nfull = _RPT // _CH
    for i in range(nfull):
        pltpu.sync_copy(sb0, accum.at[pl.ds(sid * _RPT + i * _CH, _CH)])
    rem = _RPT - nfull * _CH
    if rem:
        pltpu.sync_copy(sb0.at[pl.ds(0, rem)],
                        accum.at[pl.ds(sid * _RPT + nfull * _CH, rem)])

    plsc.subcore_barrier()

    qpg = _GRP // 4
    nquads = _NCHUNK // 4

    def step(t, carry):
        gi = t // qpg
        tl = t - gi * qpg
        base = 4 * tl

        @pl.when(tl == 0)
        def _():
            @pl.when(t > 0)
            def _():
                wait_scatter(0)
                wait_scatter(1)

            stage_group(gi)
            for j in range(4):
                pltpu.async_copy(fts_hbm.at[src_g.at[j]], gbufs[j],
                                 gsems[j])

        for j in range(4):
            c = base + j
            wait_gather(j)

            if j < 2:
                @pl.when(tl > 0)
                def _():
                    wait_scatter(j)
            else:
                wait_scatter(j % 2)

            _scale_chunk(gbufs[j], sbufs[j % 2], w_g, c)

            @pl.when(tl < qpg - 1)
            def _():
                pltpu.async_copy(fts_hbm.at[src_g.at[c + 4]], gbufs[j],
                                 gsems[j])

            pltpu.async_copy(sbufs[j % 2], accum.at[dst_g.at[c]],
                             ssems[j % 2], add=True)

        return carry

    lax.fori_loop(0, nquads, step, 0)

    wait_scatter(0)
    wait_scatter(1)

    plsc.subcore_barrier()

    nchunks = _N // 200
    for c in range((nchunks + _NS - 1) // _NS):
        chunk = sid + c * _NS

        @pl.when(chunk < nchunks)
        def _():
            r0 = chunk * 200
            pltpu.sync_copy(accum.at[pl.ds(r0, 200)],
                            out_hbm.at[cid, pl.ds(r0, 200)])


@functools.partial(jax.jit, static_argnames=())
def _sc_spmm(fts, srcb, dstb, wb):
    mesh = plsc.VectorSubcoreMesh(core_axis_name="c", subcore_axis_name="s",
                                  num_cores=_NC, num_subcores=_NS)
    f = pl.kernel(
        _sc_spmm_body,
        out_type=jax.ShapeDtypeStruct((_NC, _N, _H), jnp.float32),
        mesh=mesh,
        compiler_params=pltpu.CompilerParams(needs_layout_passes=False),
        scratch_types=[
            pltpu.VMEM_SHARED((_N, _H), jnp.float32),
            pltpu.VMEM((_GRP, _CH), jnp.int32),
            pltpu.VMEM((_GRP, _CH), jnp.int32),
            pltpu.VMEM((_GRP, _CH), jnp.float32),
            pltpu.VMEM((_CH, _H), jnp.float32),
            pltpu.VMEM((_CH, _H), jnp.float32),
            pltpu.VMEM((_CH, _H), jnp.float32),
            pltpu.VMEM((_CH, _H), jnp.float32),
            pltpu.VMEM((_CH, _H), jnp.float32),
            pltpu.VMEM((_CH, _H), jnp.float32),
            pltpu.SemaphoreType.DMA,
            pltpu.SemaphoreType.DMA,
            pltpu.SemaphoreType.DMA,
            pltpu.SemaphoreType.DMA,
            pltpu.SemaphoreType.DMA,
            pltpu.SemaphoreType.DMA,
        ],
    )
    return f(fts, srcb, dstb, wb)


def kernel(seq, edge_index, edge_weight, W0, b0, a0, W1, b1, a1, W2, b2, a2,
           sparse=1):
    x = seq[0]
    src = edge_index[0]
    dst = edge_index[1]

    pad = _EPAD - _E
    zi = jnp.zeros((pad,), jnp.int32)
    srcb = jnp.concatenate([src, zi]).reshape(_NW, _NCHUNK, _CH)
    dstb = jnp.concatenate([dst, zi]).reshape(_NW, _NCHUNK, _CH)
    wb = jnp.concatenate([edge_weight, jnp.zeros((pad,), jnp.float32)]
                         ).reshape(_NW, _NCHUNK, _CH)

    a0 = jnp.asarray(a0, jnp.float32)
    a1 = jnp.asarray(a1, jnp.float32)
    a2 = jnp.asarray(a2, jnp.float32)

    y = _tc_linear0(x, W0)
    p = _sc_spmm(y, srcb, dstb, wb)
    y = _tc_fuse(p, b0, a0, W1)
    p = _sc_spmm(y, srcb, dstb, wb)
    y = _tc_fuse(p, b1, a1, W2)
    p = _sc_spmm(y, srcb, dstb, wb)
    out = _tc_final(p, b2, a2)
    return out[None]
